# baseline (speedup 1.0000x reference)
import jax
import jax.numpy as jnp
from jax import lax
from jax.experimental import pallas as pl
from jax.experimental.pallas import tpu as pltpu
from jax.experimental.pallas import tpu_sc as plsc

N = 10000
E = 320000
D = 128
L = 3
H = (L * D) // 2

NC, NS = 2, 16
NPAD = N + 112
EPT = E // (NC * NS)
CHUNK = 40
NCHUNK = 256
EPTP = NCHUNK * CHUNK
IBLK = 16
NBLK = NCHUNK // IBLK
RING = 8
STRIP = NPAD // NS

_MESH = dict(mesh=plsc.VectorSubcoreMesh(core_axis_name="c", subcore_axis_name="s"))


def _leaky(x):
    return jnp.where(x > 0, x, 0.1 * x)


def _deg_body(srcf, dstf, degP, idxs_v, idxd_v, degs_v, degd_v):
    c = lax.axis_index("c")
    s = lax.axis_index("s")
    zero16 = jnp.zeros((16,), jnp.float32)
    one16 = jnp.ones((16,), jnp.float32)

    @pl.loop(0, NPAD // 16)
    def _zero(i):
        degs_v[pl.ds(i * 16, 16)] = zero16
        degd_v[pl.ds(i * 16, 16)] = zero16

    pltpu.sync_copy(srcf.at[c, s], idxs_v)
    pltpu.sync_copy(dstf.at[c, s], idxd_v)

    @pl.loop(0, EPTP // 16)
    def _hist(i):
        vs = idxs_v[pl.ds(i * 16, 16)]
        vd = idxd_v[pl.ds(i * 16, 16)]
        plsc.addupdate_scatter(degs_v, [vs], one16)
        plsc.addupdate_scatter(degd_v, [vd], one16)

    pltpu.sync_copy(degs_v, degP.at[c, s, 0])
    pltpu.sync_copy(degd_v, degP.at[c, s, 1])


_deg_call = pl.kernel(
    _deg_body,
    out_type=jax.ShapeDtypeStruct((NC, NS, 2, NPAD), jnp.float32),
    scratch_types=[
        pltpu.VMEM((EPTP,), jnp.int32),
        pltpu.VMEM((EPTP,), jnp.int32),
        pltpu.VMEM((NPAD,), jnp.float32),
        pltpu.VMEM((NPAD,), jnp.float32),
    ],
    compiler_params=pltpu.CompilerParams(needs_layout_passes=False),
    **_MESH,
)


def _edge_body(hpad, srcr, dstr, aggP, sidx, didx, gbuf, acc,
               gsem0, gsem1, gsem2, gsem3, gsem4, gsem5, gsem6, gsem7, isem):
    c = lax.axis_index("c")
    s = lax.axis_index("s")
    gsems = (gsem0, gsem1, gsem2, gsem3, gsem4, gsem5, gsem6, gsem7)

    base = s * STRIP
    for k in range(STRIP // (NPAD - N)):
        pltpu.sync_copy(hpad.at[pl.ds(N, NPAD - N)],
                        acc.at[pl.ds(base + k * (NPAD - N), NPAD - N)])
    rem = STRIP % (NPAD - N)
    if rem:
        pltpu.sync_copy(hpad.at[pl.ds(N, rem)],
                        acc.at[pl.ds(base + STRIP - rem, rem)])
    plsc.subcore_barrier()

    pltpu.sync_copy(srcr.at[c, s, pl.ds(0, IBLK)], sidx.at[0])
    pltpu.sync_copy(dstr.at[c, s, pl.ds(0, IBLK)], didx.at[0])
    pltpu.async_copy(srcr.at[c, s, pl.ds(IBLK, IBLK)], sidx.at[1], isem)
    pltpu.async_copy(dstr.at[c, s, pl.ds(IBLK, IBLK)], didx.at[1], isem)
    for bb in range(RING):
        pltpu.async_copy(hpad.at[sidx.at[0, bb]], gbuf.at[bb], gsems[bb])

    @pl.loop(0, NBLK // 2)
    def _outer(t):
        for kk in range(2):
            k = t * 2 + kk
            nb = (kk + 1) % 2
            for i in range(IBLK):
                j = k * IBLK + i
                b = i % RING
                if i == 0:
                    cond = (k + 1 < NBLK) if kk == 1 else (
                        (t > 0) & (k + 1 < NBLK))

                    @pl.when(cond)
                    def _stage():
                        pltpu.async_copy(
                            srcr.at[c, s, pl.ds((k + 1) * IBLK, IBLK)],
                            sidx.at[nb], isem)
                        pltpu.async_copy(
                            dstr.at[c, s, pl.ds((k + 1) * IBLK, IBLK)],
                            didx.at[nb], isem)
                if i == IBLK - RING:
                    @pl.when(k + 1 < NBLK)
                    def _wait_idx():
                        pltpu.make_async_copy(
                            srcr.at[c, s, pl.ds(0, IBLK)], sidx.at[nb],
                            isem).wait()
                        pltpu.make_async_copy(
                            dstr.at[c, s, pl.ds(0, IBLK)], didx.at[nb],
                            isem).wait()
                pltpu.make_async_copy(hpad.at[sidx.at[kk, i]], gbuf.at[b],
                                      gsems[b]).wait()
                pltpu.sync_copy(gbuf.at[b], acc.at[didx.at[kk, i]], add=True)

                @pl.when(j + RING < NCHUNK)
                def _next():
                    if i + RING < IBLK:
                        nidx = sidx.at[kk, i + RING]
                    else:
                        nidx = sidx.at[nb, i + RING - IBLK]
                    pltpu.async_copy(hpad.at[nidx], gbuf.at[b], gsems[b])

    plsc.subcore_barrier()
    pltpu.sync_copy(acc.at[pl.ds(s * STRIP, STRIP)],
                    aggP.at[c, pl.ds(s * STRIP, STRIP)])


_edge_call = pl.kernel(
    _edge_body,
    out_type=jax.ShapeDtypeStruct((NC, NPAD, D), jnp.float32),
    scratch_types=[
        pltpu.VMEM((2, IBLK, CHUNK), jnp.int32),
        pltpu.VMEM((2, IBLK, CHUNK), jnp.int32),
        pltpu.VMEM((RING, CHUNK, D), jnp.float32),
        pltpu.VMEM_SHARED((NPAD, D), jnp.float32),
    ] + [pltpu.SemaphoreType.DMA] * (RING + 1),
    **_MESH,
)


def _prep_body(x_ref, degP_ref, hpad_ref, ns_ref, nd_ref):
    degP = jnp.sum(degP_ref[...], axis=(0, 1))
    deg_s = degP[0, :N]
    deg_d = degP[1, :N]
    ns = lax.rsqrt(jnp.maximum(deg_s, 1.0)).reshape(N, 1)
    nd = lax.rsqrt(jnp.maximum(deg_d, 1.0)).reshape(N, 1)
    ns_ref[...] = ns
    nd_ref[...] = nd
    hpad_ref[pl.ds(0, N), :] = x_ref[...] * ns
    hpad_ref[pl.ds(N, NPAD - N), :] = jnp.zeros((NPAD - N, D), jnp.float32)


@jax.jit
def _prep_call(x, degP):
    return pl.pallas_call(
        _prep_body,
        out_shape=(
            jax.ShapeDtypeStruct((NPAD, D), jnp.float32),
            jax.ShapeDtypeStruct((N, 1), jnp.float32),
            jax.ShapeDtypeStruct((N, 1), jnp.float32),
        ),
    )(x, degP)


def _dense_body(aggP_ref, nd_ref, ns_ref, w_ref, gamma_ref, beta_ref,
                lam_ref, emb_ref, hnext_ref):
    eps = 1e-5
    agg = aggP_ref[0, :N, :] + aggP_ref[1, :N, :]
    z = jnp.dot(agg * nd_ref[...], w_ref[...],
                preferred_element_type=jnp.float32)

    lam = lam_ref[...]
    lam_m = jnp.max(lam, axis=0, keepdims=True)
    ew = jnp.exp(lam - lam_m)
    w = ew / jnp.sum(ew, axis=0, keepdims=True)

    mu_n = jnp.mean(z, axis=1, keepdims=True)
    var_n = jnp.mean((z - mu_n) ** 2, axis=1, keepdims=True)
    xn_node = (z - mu_n) * lax.rsqrt(var_n + eps)

    mu_g = jnp.mean(z, axis=0, keepdims=True)
    var_g = jnp.mean((z - mu_g) ** 2, axis=0, keepdims=True)
    xn_graph = (z - mu_g) * lax.rsqrt(var_g + eps)

    ms = jnp.mean(z * z, axis=0, keepdims=True)
    xn_rms = z * lax.rsqrt(ms + eps)

    mix = w[0:1] * xn_node + w[1:2] * xn_graph + w[2:3] * xn_rms
    y = _leaky(gamma_ref[...] * mix + beta_ref[...])
    emb_ref[...] = jnp.mean(y, axis=0, keepdims=True)
    hnext_ref[pl.ds(0, N), :] = y * ns_ref[...]
    hnext_ref[pl.ds(N, NPAD - N), :] = jnp.zeros((NPAD - N, D), jnp.float32)


@jax.jit
def _dense_call(aggP, nd, ns, w, gamma, beta, lam):
    return pl.pallas_call(
        _dense_body,
        out_shape=(
            jax.ShapeDtypeStruct((1, D), jnp.float32),
            jax.ShapeDtypeStruct((NPAD, D), jnp.float32),
        ),
    )(aggP, nd, ns, w, gamma.reshape(1, D), beta.reshape(1, D), lam)


def _lstm_body(seq_ref, wih_f_ref, whh_f_ref, b_f_ref, wih_b_ref, whh_b_ref,
               b_b_ref, att_w_ref, att_b_ref, out_ref):
    seq = seq_ref[...]

    def run_dir(x_steps, wih, whh, b):
        h = jnp.zeros((1, H), jnp.float32)
        c = jnp.zeros((1, H), jnp.float32)
        ys = []
        for x in x_steps:
            gates = (jnp.dot(x, wih.T, preferred_element_type=jnp.float32)
                     + jnp.dot(h, whh.T, preferred_element_type=jnp.float32)
                     + b)
            i = jax.nn.sigmoid(gates[:, 0:H])
            f = jax.nn.sigmoid(gates[:, H:2 * H])
            g = jnp.tanh(gates[:, 2 * H:3 * H])
            o = jax.nn.sigmoid(gates[:, 3 * H:4 * H])
            c = f * c + i * g
            h = o * jnp.tanh(c)
            ys.append(h)
        return ys

    steps = [seq[l:l + 1] for l in range(L)]
    ys_f = run_dir(steps, wih_f_ref[...], whh_f_ref[...], b_f_ref[...])
    ys_b = run_dir(steps[::-1], wih_b_ref[...], whh_b_ref[...],
                   b_b_ref[...])[::-1]
    ys = jnp.concatenate(
        [jnp.concatenate([f, b], axis=1) for f, b in zip(ys_f, ys_b)], axis=0)
    scores = jnp.dot(ys, att_w_ref[...], preferred_element_type=jnp.float32)
    scores = scores + att_b_ref[...]
    s = scores[:, 0]
    alpha = jax.nn.softmax(s - jnp.max(s), axis=0)
    jk = jnp.sum(alpha[:, None] * seq, axis=0, keepdims=True)
    out_ref[...] = _leaky(jk)


@jax.jit
def _lstm_readout(seq, wih_f, whh_f, b_f, wih_b, whh_b, b_b, att_w, att_b):
    return pl.pallas_call(
        _lstm_body,
        out_shape=jax.ShapeDtypeStruct((1, D), jnp.float32),
    )(seq, wih_f, whh_f, b_f.reshape(1, 4 * H), wih_b, whh_b,
      b_b.reshape(1, 4 * H), att_w, att_b.reshape(1, 1))


def kernel(node_feats, edge_index, W, norm_gamma, norm_beta, norm_lambdas,
           wih_f, whh_f, b_f, wih_b, whh_b, b_b, att_w, att_b):
    src = edge_index[0].astype(jnp.int32)
    dst = edge_index[1].astype(jnp.int32)
    pad = ((0, 0), (0, 0), (0, EPTP - EPT))
    srcr = jnp.pad(src.reshape(NC, NS, EPT), pad, constant_values=N)
    dstr = jnp.pad(dst.reshape(NC, NS, EPT), pad, constant_values=N)
    src4 = srcr.reshape(NC, NS, NCHUNK, CHUNK)
    dst4 = dstr.reshape(NC, NS, NCHUNK, CHUNK)
    degP = _deg_call(srcr, dstr)
    hpad, ns, nd = _prep_call(node_feats, degP)

    embs = []
    for l in range(L):
        aggP = _edge_call(hpad, src4, dst4)
        emb, hpad = _dense_call(aggP, nd, ns, W[l], norm_gamma[l],
                                norm_beta[l], norm_lambdas[l])
        embs.append(emb)
    seq = jnp.concatenate(embs, axis=0)
    return _lstm_readout(seq, wih_f, whh_f, b_f, wih_b, whh_b, b_b,
                         att_w, att_b)


# speedup vs baseline: 1.0011x; 1.0011x over previous
import jax
import jax.numpy as jnp
from jax import lax
from jax.experimental import pallas as pl
from jax.experimental.pallas import tpu as pltpu
from jax.experimental.pallas import tpu_sc as plsc

N = 10000
E = 320000
D = 128
L = 3
H = (L * D) // 2

NC, NS = 2, 16
NPAD = N + 112
EPT = E // (NC * NS)
CHUNK = 80
NCHUNK = 128
EPTP = NCHUNK * CHUNK
IBLK = 8
NBLK = NCHUNK // IBLK
RING = 4
STRIP = NPAD // NS

_MESH = dict(mesh=plsc.VectorSubcoreMesh(core_axis_name="c", subcore_axis_name="s"))


def _leaky(x):
    return jnp.where(x > 0, x, 0.1 * x)


def _deg_body(srcf, dstf, degP, idxs_v, idxd_v, degs_v, degd_v):
    c = lax.axis_index("c")
    s = lax.axis_index("s")
    zero16 = jnp.zeros((16,), jnp.float32)
    one16 = jnp.ones((16,), jnp.float32)

    @pl.loop(0, NPAD // 16)
    def _zero(i):
        degs_v[pl.ds(i * 16, 16)] = zero16
        degd_v[pl.ds(i * 16, 16)] = zero16

    pltpu.sync_copy(srcf.at[c, s], idxs_v)
    pltpu.sync_copy(dstf.at[c, s], idxd_v)

    @pl.loop(0, EPTP // 16)
    def _hist(i):
        vs = idxs_v[pl.ds(i * 16, 16)]
        vd = idxd_v[pl.ds(i * 16, 16)]
        plsc.addupdate_scatter(degs_v, [vs], one16)
        plsc.addupdate_scatter(degd_v, [vd], one16)

    pltpu.sync_copy(degs_v, degP.at[c, s, 0])
    pltpu.sync_copy(degd_v, degP.at[c, s, 1])


_deg_call = pl.kernel(
    _deg_body,
    out_type=jax.ShapeDtypeStruct((NC, NS, 2, NPAD), jnp.float32),
    scratch_types=[
        pltpu.VMEM((EPTP,), jnp.int32),
        pltpu.VMEM((EPTP,), jnp.int32),
        pltpu.VMEM((NPAD,), jnp.float32),
        pltpu.VMEM((NPAD,), jnp.float32),
    ],
    compiler_params=pltpu.CompilerParams(needs_layout_passes=False),
    **_MESH,
)


def _edge_body(hpad, srcr, dstr, aggP, sidx, didx, gbuf, acc,
               gsem0, gsem1, gsem2, gsem3, isem):
    c = lax.axis_index("c")
    s = lax.axis_index("s")
    gsems = (gsem0, gsem1, gsem2, gsem3)

    base = s * STRIP
    for k in range(STRIP // (NPAD - N)):
        pltpu.sync_copy(hpad.at[pl.ds(N, NPAD - N)],
                        acc.at[pl.ds(base + k * (NPAD - N), NPAD - N)])
    rem = STRIP % (NPAD - N)
    if rem:
        pltpu.sync_copy(hpad.at[pl.ds(N, rem)],
                        acc.at[pl.ds(base + STRIP - rem, rem)])
    plsc.subcore_barrier()

    pltpu.sync_copy(srcr.at[c, s, pl.ds(0, IBLK)], sidx.at[0])
    pltpu.sync_copy(dstr.at[c, s, pl.ds(0, IBLK)], didx.at[0])
    pltpu.async_copy(srcr.at[c, s, pl.ds(IBLK, IBLK)], sidx.at[1], isem)
    pltpu.async_copy(dstr.at[c, s, pl.ds(IBLK, IBLK)], didx.at[1], isem)
    for bb in range(RING):
        pltpu.async_copy(hpad.at[sidx.at[0, bb]], gbuf.at[bb], gsems[bb])

    @pl.loop(0, NBLK // 2)
    def _outer(t):
        for kk in range(2):
            k = t * 2 + kk
            nb = (kk + 1) % 2
            for i in range(IBLK):
                j = k * IBLK + i
                b = i % RING
                if i == 0:
                    cond = (k + 1 < NBLK) if kk == 1 else (
                        (t > 0) & (k + 1 < NBLK))

                    @pl.when(cond)
                    def _stage():
                        pltpu.async_copy(
                            srcr.at[c, s, pl.ds((k + 1) * IBLK, IBLK)],
                            sidx.at[nb], isem)
                        pltpu.async_copy(
                            dstr.at[c, s, pl.ds((k + 1) * IBLK, IBLK)],
                            didx.at[nb], isem)
                if i == IBLK - RING:
                    @pl.when(k + 1 < NBLK)
                    def _wait_idx():
                        pltpu.make_async_copy(
                            srcr.at[c, s, pl.ds(0, IBLK)], sidx.at[nb],
                            isem).wait()
                        pltpu.make_async_copy(
                            dstr.at[c, s, pl.ds(0, IBLK)], didx.at[nb],
                            isem).wait()
                pltpu.make_async_copy(hpad.at[sidx.at[kk, i]], gbuf.at[b],
                                      gsems[b]).wait()
                pltpu.sync_copy(gbuf.at[b], acc.at[didx.at[kk, i]], add=True)

                @pl.when(j + RING < NCHUNK)
                def _next():
                    if i + RING < IBLK:
                        nidx = sidx.at[kk, i + RING]
                    else:
                        nidx = sidx.at[nb, i + RING - IBLK]
                    pltpu.async_copy(hpad.at[nidx], gbuf.at[b], gsems[b])

    plsc.subcore_barrier()
    pltpu.sync_copy(acc.at[pl.ds(s * STRIP, STRIP)],
                    aggP.at[c, pl.ds(s * STRIP, STRIP)])


_edge_call = pl.kernel(
    _edge_body,
    out_type=jax.ShapeDtypeStruct((NC, NPAD, D), jnp.float32),
    scratch_types=[
        pltpu.VMEM((2, IBLK, CHUNK), jnp.int32),
        pltpu.VMEM((2, IBLK, CHUNK), jnp.int32),
        pltpu.VMEM((RING, CHUNK, D), jnp.float32),
        pltpu.VMEM_SHARED((NPAD, D), jnp.float32),
        pltpu.SemaphoreType.DMA,
        pltpu.SemaphoreType.DMA,
        pltpu.SemaphoreType.DMA,
        pltpu.SemaphoreType.DMA,
        pltpu.SemaphoreType.DMA,
    ],
    **_MESH,
)


def _prep_body(x_ref, degP_ref, hpad_ref, ns_ref, nd_ref):
    degP = jnp.sum(degP_ref[...], axis=(0, 1))
    deg_s = degP[0, :N]
    deg_d = degP[1, :N]
    ns = lax.rsqrt(jnp.maximum(deg_s, 1.0)).reshape(N, 1)
    nd = lax.rsqrt(jnp.maximum(deg_d, 1.0)).reshape(N, 1)
    ns_ref[...] = ns
    nd_ref[...] = nd
    hpad_ref[pl.ds(0, N), :] = x_ref[...] * ns
    hpad_ref[pl.ds(N, NPAD - N), :] = jnp.zeros((NPAD - N, D), jnp.float32)


@jax.jit
def _prep_call(x, degP):
    return pl.pallas_call(
        _prep_body,
        out_shape=(
            jax.ShapeDtypeStruct((NPAD, D), jnp.float32),
            jax.ShapeDtypeStruct((N, 1), jnp.float32),
            jax.ShapeDtypeStruct((N, 1), jnp.float32),
        ),
    )(x, degP)


def _dense_body(aggP_ref, nd_ref, ns_ref, w_ref, gamma_ref, beta_ref,
                lam_ref, emb_ref, hnext_ref):
    eps = 1e-5
    agg = aggP_ref[0, :N, :] + aggP_ref[1, :N, :]
    z = jnp.dot(agg * nd_ref[...], w_ref[...],
                preferred_element_type=jnp.float32)

    lam = lam_ref[...]
    lam_m = jnp.max(lam, axis=0, keepdims=True)
    ew = jnp.exp(lam - lam_m)
    w = ew / jnp.sum(ew, axis=0, keepdims=True)

    mu_n = jnp.mean(z, axis=1, keepdims=True)
    var_n = jnp.mean((z - mu_n) ** 2, axis=1, keepdims=True)
    xn_node = (z - mu_n) * lax.rsqrt(var_n + eps)

    mu_g = jnp.mean(z, axis=0, keepdims=True)
    var_g = jnp.mean((z - mu_g) ** 2, axis=0, keepdims=True)
    xn_graph = (z - mu_g) * lax.rsqrt(var_g + eps)

    ms = jnp.mean(z * z, axis=0, keepdims=True)
    xn_rms = z * lax.rsqrt(ms + eps)

    mix = w[0:1] * xn_node + w[1:2] * xn_graph + w[2:3] * xn_rms
    y = _leaky(gamma_ref[...] * mix + beta_ref[...])
    emb_ref[...] = jnp.mean(y, axis=0, keepdims=True)
    hnext_ref[pl.ds(0, N), :] = y * ns_ref[...]
    hnext_ref[pl.ds(N, NPAD - N), :] = jnp.zeros((NPAD - N, D), jnp.float32)


@jax.jit
def _dense_call(aggP, nd, ns, w, gamma, beta, lam):
    return pl.pallas_call(
        _dense_body,
        out_shape=(
            jax.ShapeDtypeStruct((1, D), jnp.float32),
            jax.ShapeDtypeStruct((NPAD, D), jnp.float32),
        ),
    )(aggP, nd, ns, w, gamma.reshape(1, D), beta.reshape(1, D), lam)


def _lstm_body(seq_ref, wih_f_ref, whh_f_ref, b_f_ref, wih_b_ref, whh_b_ref,
               b_b_ref, att_w_ref, att_b_ref, out_ref):
    seq = seq_ref[...]

    def run_dir(x_steps, wih, whh, b):
        h = jnp.zeros((1, H), jnp.float32)
        c = jnp.zeros((1, H), jnp.float32)
        ys = []
        for x in x_steps:
            gates = (jnp.dot(x, wih.T, preferred_element_type=jnp.float32)
                     + jnp.dot(h, whh.T, preferred_element_type=jnp.float32)
                     + b)
            i = jax.nn.sigmoid(gates[:, 0:H])
            f = jax.nn.sigmoid(gates[:, H:2 * H])
            g = jnp.tanh(gates[:, 2 * H:3 * H])
            o = jax.nn.sigmoid(gates[:, 3 * H:4 * H])
            c = f * c + i * g
            h = o * jnp.tanh(c)
            ys.append(h)
        return ys

    steps = [seq[l:l + 1] for l in range(L)]
    ys_f = run_dir(steps, wih_f_ref[...], whh_f_ref[...], b_f_ref[...])
    ys_b = run_dir(steps[::-1], wih_b_ref[...], whh_b_ref[...],
                   b_b_ref[...])[::-1]
    ys = jnp.concatenate(
        [jnp.concatenate([f, b], axis=1) for f, b in zip(ys_f, ys_b)], axis=0)
    scores = jnp.dot(ys, att_w_ref[...], preferred_element_type=jnp.float32)
    scores = scores + att_b_ref[...]
    s = scores[:, 0]
    alpha = jax.nn.softmax(s - jnp.max(s), axis=0)
    jk = jnp.sum(alpha[:, None] * seq, axis=0, keepdims=True)
    out_ref[...] = _leaky(jk)


@jax.jit
def _lstm_readout(seq, wih_f, whh_f, b_f, wih_b, whh_b, b_b, att_w, att_b):
    return pl.pallas_call(
        _lstm_body,
        out_shape=jax.ShapeDtypeStruct((1, D), jnp.float32),
    )(seq, wih_f, whh_f, b_f.reshape(1, 4 * H), wih_b, whh_b,
      b_b.reshape(1, 4 * H), att_w, att_b.reshape(1, 1))


def kernel(node_feats, edge_index, W, norm_gamma, norm_beta, norm_lambdas,
           wih_f, whh_f, b_f, wih_b, whh_b, b_b, att_w, att_b):
    src = edge_index[0].astype(jnp.int32)
    dst = edge_index[1].astype(jnp.int32)
    pad = ((0, 0), (0, 0), (0, EPTP - EPT))
    srcr = jnp.pad(src.reshape(NC, NS, EPT), pad, constant_values=N)
    dstr = jnp.pad(dst.reshape(NC, NS, EPT), pad, constant_values=N)
    src4 = srcr.reshape(NC, NS, NCHUNK, CHUNK)
    dst4 = dstr.reshape(NC, NS, NCHUNK, CHUNK)
    degP = _deg_call(srcr, dstr)
    hpad, ns, nd = _prep_call(node_feats, degP)

    embs = []
    for l in range(L):
        aggP = _edge_call(hpad, src4, dst4)
        emb, hpad = _dense_call(aggP, nd, ns, W[l], norm_gamma[l],
                                norm_beta[l], norm_lambdas[l])
        embs.append(emb)
    seq = jnp.concatenate(embs, axis=0)
    return _lstm_readout(seq, wih_f, whh_f, b_f, wih_b, whh_b, b_b,
                         att_w, att_b)
